# baseline (speedup 1.0000x reference)
import jax
import jax.numpy as jnp
from jax import lax
from jax.experimental import pallas as pl
from jax.experimental.pallas import tpu as pltpu

D_MODEL = 1024
BATCH = 16
NSEM = 4


def _body(idx_smem, hs_hbm, out_vmem, buf, sems):
    for b in range(BATCH):
        pltpu.make_async_copy(
            hs_hbm.at[pl.ds(idx_smem[b], 1)], buf.at[pl.ds(b, 1)], sems.at[b % NSEM]
        ).start()
    for q in range(NSEM):
        pltpu.make_async_copy(
            hs_hbm.at[pl.ds(0, BATCH // NSEM)],
            buf.at[pl.ds(0, BATCH // NSEM)],
            sems.at[q],
        ).wait()
    x = buf[...]
    ss = jnp.sum(x * x, axis=1, keepdims=True)
    out_vmem[...] = x * lax.rsqrt(jnp.maximum(ss, 1e-24))


@jax.jit
def _pooler(hs, idx):
    return pl.pallas_call(
        _body,
        in_specs=[
            pl.BlockSpec(memory_space=pltpu.MemorySpace.SMEM),
            pl.BlockSpec(memory_space=pltpu.MemorySpace.HBM),
        ],
        out_specs=pl.BlockSpec(memory_space=pltpu.MemorySpace.VMEM),
        scratch_shapes=[
            pltpu.VMEM((BATCH, D_MODEL), jnp.float32),
            pltpu.SemaphoreType.DMA((NSEM,)),
        ],
        out_shape=jax.ShapeDtypeStruct((BATCH, D_MODEL), jnp.float32),
    )(idx, hs)


def kernel(hidden_states, last_token_indices):
    hs = hidden_states.astype(jnp.float32)
    idx = last_token_indices.astype(jnp.int32)
    return _pooler(hs, idx)


# speedup vs baseline: 1.0292x; 1.0292x over previous
import jax
import jax.numpy as jnp
from jax import lax
from jax.experimental import pallas as pl
from jax.experimental.pallas import tpu as pltpu

D_MODEL = 1024
BATCH = 16


def _body(idx_smem, hs_hbm, out_vmem, buf, sem):
    for b in range(BATCH):
        pltpu.make_async_copy(
            hs_hbm.at[pl.ds(idx_smem[b], 1)], buf.at[pl.ds(b, 1)], sem
        ).start()
    pltpu.make_async_copy(hs_hbm.at[pl.ds(0, BATCH)], buf, sem).wait()
    x = buf[...]
    ss = jnp.sum(x * x, axis=1, keepdims=True)
    out_vmem[...] = x * lax.rsqrt(jnp.maximum(ss, 1e-24))


@jax.jit
def _pooler(hs, idx):
    return pl.pallas_call(
        _body,
        in_specs=[
            pl.BlockSpec(memory_space=pltpu.MemorySpace.SMEM),
            pl.BlockSpec(memory_space=pltpu.MemorySpace.HBM),
        ],
        out_specs=pl.BlockSpec(memory_space=pltpu.MemorySpace.VMEM),
        scratch_shapes=[
            pltpu.VMEM((BATCH, D_MODEL), jnp.float32),
            pltpu.SemaphoreType.DMA,
        ],
        out_shape=jax.ShapeDtypeStruct((BATCH, D_MODEL), jnp.float32),
    )(idx, hs)


def kernel(hidden_states, last_token_indices):
    hs = hidden_states.astype(jnp.float32)
    idx = last_token_indices.astype(jnp.int32)
    return _pooler(hs, idx)
